# baseline (speedup 1.0000x reference)
import functools

import jax
import jax.numpy as jnp
from jax import lax
from jax.experimental import pallas as pl
from jax.experimental.pallas import tpu as pltpu
from jax.experimental.pallas import tpu_sc as plsc

ROWS = 128
VOCAB = 100000
L = 16
NC = 2
NS = 16
NW = NC * NS
RPW = ROWS // NW

SC_COLS = 99840
CHUNK = 32768
NFULL = SC_COLS // CHUNK
TAIL = SC_COLS - NFULL * CHUNK
NST = 4
GROUP = NST * L

INT_MAX = 2**31 - 1
NEG_INF = float("-inf")


def _fold_chunk(buf, base, n_groups, states):
    @plsc.parallel_loop(0, n_groups, step=1, unroll=4, carry=states)
    def body(g, states):
        out = []
        for q in range(NST):
            vmax, vpos = states[q]
            off = g * GROUP + q * L
            v = buf[pl.ds(off, L)]
            better = v > vmax
            p = base + off
            vpos = jnp.where(better, p, vpos)
            vmax = jnp.where(better, v, vmax)
            out.append((vmax, vpos))
        return tuple(out)

    return body


def _sc_argmax_kernel(x_hbm, out_val_hbm, out_idx_hbm, buf0, buf1,
                      val_buf, idx_buf, red_v_ref, red_g_ref, sem0, sem1):
    wid = lax.axis_index("s") * NC + lax.axis_index("c")
    bufs = (buf0, buf1)
    sems = (sem0, sem1)
    iota = lax.iota(jnp.int32, L)

    sizes = [CHUNK] * NFULL + [TAIL]
    offs = [c * CHUNK for c in range(NFULL + 1)]
    transfers = [(r, c) for r in range(RPW) for c in range(NFULL + 1)]
    ntr = len(transfers)

    def start(t):
        r, c = transfers[t]
        row = wid * RPW + r
        return pltpu.async_copy(
            x_hbm.at[row, pl.ds(offs[c], sizes[c])],
            bufs[t % 2].at[pl.ds(0, sizes[c])],
            sems[t % 2])

    copies = {0: start(0)}
    states = None
    for t in range(ntr):
        r, c = transfers[t]
        if t + 1 < ntr:
            copies[t + 1] = start(t + 1)
        copies[t].wait()
        buf = bufs[t % 2]
        if c == 0:
            states = tuple((jnp.full((L,), NEG_INF, jnp.float32),
                            jnp.full((L,), 0, jnp.int32))
                           for _ in range(NST))
        states = _fold_chunk(buf, c * CHUNK, sizes[c] // GROUP, states)
        if c == NFULL:
            mv, mg = states[0][0], states[0][1] + iota
            for q in range(1, NST):
                vq, gq = states[q][0], states[q][1] + iota
                better = (vq > mv) | ((vq == mv) & (gq < mg))
                mg = jnp.where(better, gq, mg)
                mv = jnp.where(better, vq, mv)
            for step in (8, 4, 2, 1):
                red_v_ref[...] = mv
                red_g_ref[...] = mg
                pidx = iota ^ step
                vv = plsc.load_gather(red_v_ref, [pidx])
                gg = plsc.load_gather(red_g_ref, [pidx])
                better = (vv > mv) | ((vv == mv) & (gg < mg))
                mg = jnp.where(better, gg, mg)
                mv = jnp.where(better, vv, mv)
            for j in range(8):
                val_buf[pl.ds(j * L, L)] = mv
                idx_buf[pl.ds(j * L, L)] = mg
            row = wid * RPW + r
            pltpu.sync_copy(val_buf, out_val_hbm.at[row])
            pltpu.sync_copy(idx_buf, out_idx_hbm.at[row])


@jax.jit
def _argmax_sc(m_logits):
    mesh = plsc.VectorSubcoreMesh(core_axis_name="c", subcore_axis_name="s")
    k = functools.partial(
        pl.kernel,
        mesh=mesh,
        compiler_params=pltpu.CompilerParams(needs_layout_passes=False),
        out_type=(jax.ShapeDtypeStruct((ROWS, 128), jnp.float32),
                  jax.ShapeDtypeStruct((ROWS, 128), jnp.int32)),
        scratch_types=[
            pltpu.VMEM((CHUNK,), jnp.float32),
            pltpu.VMEM((CHUNK,), jnp.float32),
            pltpu.VMEM((128,), jnp.float32),
            pltpu.VMEM((128,), jnp.int32),
            pltpu.VMEM((L,), jnp.float32),
            pltpu.VMEM((L,), jnp.int32),
            pltpu.SemaphoreType.DMA,
            pltpu.SemaphoreType.DMA,
        ],
    )(_sc_argmax_kernel)
    return k(m_logits)


TC_BLK = 512
TC_BASE = SC_COLS


def _tc_merge_body(x_ref, scv_ref, sci_ref, out_ref):
    x = x_ref[...]
    gcol = jax.lax.broadcasted_iota(jnp.int32, x.shape, 1) + TC_BASE
    x = jnp.where(gcol < VOCAB, x, -jnp.inf)
    vtail = jnp.max(x, axis=1, keepdims=True)
    itail = jnp.min(jnp.where(x == vtail, gcol, INT_MAX), axis=1,
                    keepdims=True)
    sc_v = scv_ref[:, :1]
    sc_i = sci_ref[:, :1]
    out_ref[...] = jnp.where(sc_v >= vtail, sc_i, itail)


@jax.jit
def _tc_merge(m_logits, sc_val, sc_idx):
    return pl.pallas_call(
        _tc_merge_body,
        grid=(1,),
        in_specs=[
            pl.BlockSpec((ROWS, TC_BLK), lambda i: (0, TC_BASE // TC_BLK)),
            pl.BlockSpec((ROWS, 128), lambda i: (0, 0)),
            pl.BlockSpec((ROWS, 128), lambda i: (0, 0)),
        ],
        out_specs=pl.BlockSpec((ROWS, 1), lambda i: (0, 0)),
        out_shape=jax.ShapeDtypeStruct((ROWS, 1), jnp.int32),
    )(m_logits, sc_val, sc_idx)


def kernel(m_logits):
    x = m_logits.astype(jnp.float32)
    sc_val, sc_idx = _argmax_sc(x)
    token = _tc_merge(x, sc_val, sc_idx)
    return token.astype(jnp.int64)
